# baseline (speedup 1.0000x reference)
import functools
import jax, jax.numpy as jnp
from jax.experimental import pallas as pl

def _body(w_ref, out_ref):
    out_ref[...] = w_ref[:32, :128]

@functools.partial(jax.jit)
def kernel(x, W0, b0, W1, b1, W2, b2, Wp, bp):
    Wv = Wp.reshape(25000, 1024)
    o = pl.pallas_call(
        _body,
        grid=(25,),
        in_specs=[pl.BlockSpec((1000, 1024), lambda i: (i, 0))],
        out_specs=pl.BlockSpec((32, 128), lambda i: (0, 0)),
        out_shape=jax.ShapeDtypeStruct((32, 128), jnp.float32),
    )(Wv)
    return jnp.broadcast_to(o[:, :1], (32, 100000)) + 0.0


# speedup vs baseline: 2.3349x; 2.3349x over previous
import functools

import jax
import jax.numpy as jnp
from jax.experimental import pallas as pl
from jax.experimental.pallas import tpu as pltpu

D_IN_ = 16384
EMB_ = 256
N_CLASSES_ = 100000
BATCH_ = 32

K_CHUNK = 2048
N_TILE = 8192
N_TILES = (N_CLASSES_ + N_TILE - 1) // N_TILE
S_TILE = 12288
S_TILES = (N_CLASSES_ + S_TILE - 1) // S_TILE


def _embrace_body(x_ref, w0_ref, w1_ref, w2_ref, b0_ref, b1_ref, b2_ref,
                  mask_ref, out_ref, acc_ref):
    k = pl.program_id(0)
    nk = pl.num_programs(0)

    @pl.when(k == 0)
    def _init():
        acc_ref[...] = (mask_ref[0] * b0_ref[...]
                        + mask_ref[1] * b1_ref[...]
                        + mask_ref[2] * b2_ref[...])

    acc = acc_ref[...]
    acc += mask_ref[0] * jnp.dot(x_ref[0], w0_ref[...],
                                 preferred_element_type=jnp.float32)
    acc += mask_ref[1] * jnp.dot(x_ref[1], w1_ref[...],
                                 preferred_element_type=jnp.float32)
    acc += mask_ref[2] * jnp.dot(x_ref[2], w2_ref[...],
                                 preferred_element_type=jnp.float32)
    acc_ref[...] = acc

    @pl.when(k == nk - 1)
    def _emit():
        out_ref[...] = acc_ref[...]


def _logits_body(emb_ref, wp_ref, bp_ref, logits_ref, lse_ref, m_ref, s_ref):
    i = pl.program_id(0)

    logits = jnp.dot(emb_ref[...], wp_ref[...],
                     preferred_element_type=jnp.float32) + bp_ref[...]
    rem = N_CLASSES_ - i * N_TILE
    cols = jax.lax.broadcasted_iota(jnp.int32, logits.shape, 1)
    logits = jnp.where(cols < rem, logits, -jnp.inf)
    logits_ref[...] = logits
    tmax = jnp.max(logits, axis=1, keepdims=True)

    @pl.when(i == 0)
    def _first():
        m_ref[...] = tmax
        s_ref[...] = jnp.sum(jnp.exp(logits - tmax), axis=1, keepdims=True)

    @pl.when(i > 0)
    def _rest():
        m_old = m_ref[...]
        m_new = jnp.maximum(m_old, tmax)
        s_ref[...] = (s_ref[...] * jnp.exp(m_old - m_new)
                      + jnp.sum(jnp.exp(logits - m_new), axis=1,
                                keepdims=True))
        m_ref[...] = m_new

    @pl.when(i == N_TILES - 1)
    def _finish():
        lse_ref[...] = jnp.broadcast_to(m_ref[...] + jnp.log(s_ref[...]),
                                        lse_ref.shape)


def _sub_body(logits_ref, lse_ref, out_ref):
    out_ref[...] = logits_ref[...] - lse_ref[:, :1]


@functools.partial(jax.jit, static_argnames=())
def kernel(x, W0, b0, W1, b1, W2, b2, Wp, bp):
    avail = jnp.ones((BATCH_, 3), dtype=jnp.float32)
    prob = avail / jnp.sum(avail, axis=1, keepdims=True)
    sel_logits = jnp.broadcast_to(jnp.log(prob)[:, None, :], (BATCH_, EMB_, 3))
    idx = jax.random.categorical(jax.random.key(42), sel_logits, axis=-1)
    mask = jnp.transpose(jax.nn.one_hot(idx, 3, dtype=jnp.float32), (2, 0, 1))

    b0r = b0.reshape(1, EMB_)
    b1r = b1.reshape(1, EMB_)
    b2r = b2.reshape(1, EMB_)
    bpr = bp.reshape(1, N_CLASSES_)

    nk = D_IN_ // K_CHUNK
    embrace = pl.pallas_call(
        _embrace_body,
        grid=(nk,),
        in_specs=[
            pl.BlockSpec((3, BATCH_, K_CHUNK), lambda k: (0, 0, k)),
            pl.BlockSpec((K_CHUNK, EMB_), lambda k: (k, 0)),
            pl.BlockSpec((K_CHUNK, EMB_), lambda k: (k, 0)),
            pl.BlockSpec((K_CHUNK, EMB_), lambda k: (k, 0)),
            pl.BlockSpec((1, EMB_), lambda k: (0, 0)),
            pl.BlockSpec((1, EMB_), lambda k: (0, 0)),
            pl.BlockSpec((1, EMB_), lambda k: (0, 0)),
            pl.BlockSpec((3, BATCH_, EMB_), lambda k: (0, 0, 0)),
        ],
        out_specs=pl.BlockSpec((BATCH_, EMB_), lambda k: (0, 0)),
        out_shape=jax.ShapeDtypeStruct((BATCH_, EMB_), jnp.float32),
        scratch_shapes=[pltpu.VMEM((BATCH_, EMB_), jnp.float32)],
    )(x, W0, W1, W2, b0r, b1r, b2r, mask)

    logits, lse = pl.pallas_call(
        _logits_body,
        grid=(N_TILES,),
        in_specs=[
            pl.BlockSpec((BATCH_, EMB_), lambda i: (0, 0)),
            pl.BlockSpec((EMB_, N_TILE), lambda i: (0, i)),
            pl.BlockSpec((1, N_TILE), lambda i: (0, i)),
        ],
        out_specs=[
            pl.BlockSpec((BATCH_, N_TILE), lambda i: (0, i)),
            pl.BlockSpec((BATCH_, 128), lambda i: (0, 0)),
        ],
        out_shape=[
            jax.ShapeDtypeStruct((BATCH_, N_CLASSES_), jnp.float32),
            jax.ShapeDtypeStruct((BATCH_, 128), jnp.float32),
        ],
        scratch_shapes=[
            pltpu.VMEM((BATCH_, 1), jnp.float32),
            pltpu.VMEM((BATCH_, 1), jnp.float32),
        ],
    )(embrace, Wp, bpr)

    out = pl.pallas_call(
        _sub_body,
        grid=(S_TILES,),
        in_specs=[
            pl.BlockSpec((BATCH_, S_TILE), lambda i: (0, i)),
            pl.BlockSpec((BATCH_, 128), lambda i: (0, 0)),
        ],
        out_specs=pl.BlockSpec((BATCH_, S_TILE), lambda i: (0, i)),
        out_shape=jax.ShapeDtypeStruct((BATCH_, N_CLASSES_), jnp.float32),
    )(logits, lse)

    return out
